# baseline (speedup 1.0000x reference)
import jax
import jax.numpy as jnp
from jax.experimental import pallas as pl

_B, _S, _D = 16, 4096, 64
_NH = 8
_BS = 64


def _hashv(vecs, rotations, n_buckets):
    b = vecs.shape[0]
    rot = jnp.broadcast_to(rotations, (b,) + rotations.shape[1:])
    rotated = jnp.einsum('btf,bfhi->bhti', vecs, rot)
    rotated = jnp.concatenate([rotated, -rotated], axis=-1)
    buckets = jnp.argmax(rotated, axis=-1)
    offsets = (jnp.arange(_NH) * n_buckets).reshape(1, -1, 1)
    buckets = (buckets + offsets).reshape(b, -1)
    return buckets


def _lob(x):
    x_extra = jnp.concatenate([x[:, -1:], x[:, :-1]], axis=1)
    return jnp.concatenate([x, x_extra], axis=2)


def _ident_body(x_ref, o_ref):
    o_ref[...] = x_ref[...]


def kernel(qk, v, rotations):
    batch_size, seqlen, dim = qk.shape
    n_buckets = seqlen // _BS
    buckets = _hashv(qk, rotations, n_buckets)
    total = _NH * seqlen
    ticker = jnp.broadcast_to(jnp.arange(total)[None, :], (batch_size, total))
    buckets_and_t = seqlen * buckets + (ticker % seqlen)
    sticker = jnp.argsort(buckets_and_t, axis=-1)
    undo_sort = jnp.argsort(sticker, axis=-1)
    st = sticker % seqlen
    idx = jnp.broadcast_to(st[:, :, None], (batch_size, total, dim))
    sqk = jnp.take_along_axis(qk, idx, axis=1)
    sv = jnp.take_along_axis(v, idx, axis=1)
    chunk_size = _NH * n_buckets
    bq_t = st.reshape(batch_size, chunk_size, -1)
    bqk = sqk.reshape(batch_size, chunk_size, -1, dim)
    bv = sv.reshape(batch_size, chunk_size, -1, dim)
    bq = bqk
    bk = bqk / jnp.maximum(jnp.linalg.norm(bqk, axis=-1, keepdims=True), 1e-12)
    bk = _lob(bk)
    bv = _lob(bv)
    bkv_t2 = _lob(bq_t)
    dots = jnp.einsum('bhie,bhje->bhij', bq, bk) * (dim ** -0.5)
    mask = bq_t[:, :, :, None] < jnp.minimum(bkv_t2[:, :, None, :], seqlen - 1)
    dots = jnp.where(mask, -1e9, dots)
    self_mask = bq_t[:, :, :, None] == bkv_t2[:, :, None, :]
    dots = jnp.where(self_mask, -1e5, dots)
    dots_lse = jax.nn.logsumexp(dots, axis=-1, keepdims=True)
    probs = jnp.exp(dots - dots_lse)
    bo = jnp.einsum('buij,buje->buie', probs, bv)
    so = bo.reshape(batch_size, -1, dim)
    slogits = dots_lse.reshape(batch_size, -1)
    o = jnp.take_along_axis(so, jnp.broadcast_to(undo_sort[:, :, None], so.shape), axis=1)
    logits = jnp.take_along_axis(slogits, undo_sort, axis=1)
    o = o.reshape(batch_size, _NH, seqlen, dim)
    logits = logits.reshape(batch_size, _NH, seqlen, 1)
    probs2 = jnp.exp(logits - jax.nn.logsumexp(logits, axis=1, keepdims=True))
    out = jnp.sum(o * probs2, axis=1)
    out = pl.pallas_call(
        _ident_body,
        grid=(batch_size,),
        in_specs=[pl.BlockSpec((1, seqlen, dim), lambda b: (b, 0, 0))],
        out_specs=pl.BlockSpec((1, seqlen, dim), lambda b: (b, 0, 0)),
        out_shape=jax.ShapeDtypeStruct(out.shape, out.dtype),
    )(out)
    return out


# speedup vs baseline: 140.8982x; 140.8982x over previous
import functools

import jax
import jax.numpy as jnp
from jax.experimental import pallas as pl

B, S, D = 16, 4096, 64
NH = 8
BS = 64
NB = S // BS
TOT = NH * S
NCHUNK = TOT // BS


def _hashpos_body(qk_ref, rot_ref, pos_ref):
    qk = qk_ref[0]
    rot = rot_ref[...]
    rotated = jax.lax.dot_general(
        qk, rot, (((1,), (0,)), ((), ())),
        preferred_element_type=jnp.float32)

    U = 128
    NCH = S // U
    io_r = jax.lax.broadcasted_iota(jnp.int32, (U, U), 0)
    io_c = jax.lax.broadcasted_iota(jnp.int32, (U, U), 1)
    trilB = jnp.broadcast_to(
        (io_r > io_c).astype(jnp.float32)[None], (NCH, U, U))
    ioc_r = jax.lax.broadcasted_iota(jnp.int32, (NCH, NCH), 0)
    ioc_c = jax.lax.broadcasted_iota(jnp.int32, (NCH, NCH), 1)
    tril_c = (ioc_r > ioc_c).astype(jnp.float32)
    iob_r = jax.lax.broadcasted_iota(jnp.int32, (NB, NB), 0)
    iob_c = jax.lax.broadcasted_iota(jnp.int32, (NB, NB), 1)
    upper_b = (iob_r < iob_c).astype(jnp.float32)

    rows = []
    for h in range(NH):
        r = rotated[:, h * 32:(h + 1) * 32]
        c = jnp.concatenate([r, -r], axis=-1)
        m = jnp.max(c, axis=-1, keepdims=True)
        i64 = jax.lax.broadcasted_iota(jnp.int32, (S, NB), 1)
        amax = jnp.min(jnp.where(c >= m, i64, NB), axis=-1, keepdims=True)
        oh = (i64 == amax).astype(jnp.float32)
        ohc = oh.reshape(NCH, U, NB)
        sc = jnp.sum(ohc, axis=1)
        pfx = jax.lax.dot_general(
            tril_c, sc, (((1,), (0,)), ((), ())),
            preferred_element_type=jnp.float32)
        cnt = jnp.sum(sc, axis=0, keepdims=True)
        excl = jax.lax.dot_general(
            cnt, upper_b, (((1,), (0,)), ((), ())),
            preferred_element_type=jnp.float32)
        mi = jax.lax.dot_general(
            trilB, ohc, (((2,), (1,)), ((0,), (0,))),
            preferred_element_type=jnp.float32)
        f = mi + pfx[:, None, :] + excl.reshape(1, 1, NB)
        posr = jnp.sum(f * ohc, axis=-1).reshape(S)
        rows.append(posr + h * S)
    pos_all = jnp.stack(rows, axis=0)
    pos_ref[0] = pos_all.astype(jnp.int32)


def _hashpos(qk, rot2):
    return pl.pallas_call(
        _hashpos_body,
        grid=(B,),
        in_specs=[
            pl.BlockSpec((1, S, D), lambda b: (b, 0, 0)),
            pl.BlockSpec((D, NH * 32), lambda b: (0, 0)),
        ],
        out_specs=pl.BlockSpec((1, NH, S), lambda b: (b, 0, 0)),
        out_shape=jax.ShapeDtypeStruct((B, NH, S), jnp.int32),
    )(qk, rot2)


CW = 192
QBLK = 1024
SUB = 256
KW = SUB + BS


def _attn_body(cm_ref, ch_ref, so_ref, sl_ref):
    cm = cm_ref[0]
    ch = ch_ref[0]
    ext = jnp.concatenate([ch, cm], axis=0)
    ext_qk = ext[:, :D]
    ext_v = ext[:, D:2 * D]
    ext_t = ext[:, 2 * D]
    nrm = jnp.sqrt(jnp.sum(ext_qk * ext_qk, axis=-1, keepdims=True))
    ext_k = ext_qk / jnp.maximum(nrm, 1e-12)
    qm = cm[:, :D]
    qt_all = cm[:, 2 * D]
    outs, lses = [], []
    for sb in range(QBLK // SUB):
        q = qm[sb * SUB:(sb + 1) * SUB]
        qt = qt_all[sb * SUB:(sb + 1) * SUB]
        k = ext_k[sb * SUB: sb * SUB + KW]
        v = ext_v[sb * SUB: sb * SUB + KW]
        kt = ext_t[sb * SUB: sb * SUB + KW]
        dots = jax.lax.dot_general(
            q, k, (((1,), (1,)), ((), ())),
            preferred_element_type=jnp.float32) * (D ** -0.5)
        qt2 = qt[:, None]
        kt2 = kt[None, :]
        dots = jnp.where(qt2 < kt2, -1e9, dots)
        dots = jnp.where(qt2 == kt2, -1e5, dots)
        qi = jax.lax.broadcasted_iota(jnp.int32, (SUB, KW), 0)
        kj = jax.lax.broadcasted_iota(jnp.int32, (SUB, KW), 1)
        qc = qi // BS
        kc = kj // BS - 1
        allowed = (kc == qc) | (kc == qc - 1)
        dots = jnp.where(allowed, dots, -1e9)
        m = jnp.max(dots, axis=-1, keepdims=True)
        p = jnp.exp(dots - m)
        s = jnp.sum(p, axis=-1, keepdims=True)
        o = jax.lax.dot_general(
            p, v, (((1,), (0,)), ((), ())),
            preferred_element_type=jnp.float32) / s
        outs.append(o)
        lses.append(m + jnp.log(s))
    so_ref[0] = jnp.concatenate(outs, axis=0)
    sl_ref[0] = jnp.concatenate(lses, axis=0).reshape(QBLK // BS, BS)


def _attn(cmb):
    nprog = TOT // QBLK
    qb_chunks = QBLK // BS
    return pl.pallas_call(
        _attn_body,
        grid=(B, nprog),
        in_specs=[
            pl.BlockSpec((1, QBLK, CW), lambda b, i: (b, i, 0)),
            pl.BlockSpec((1, BS, CW),
                         lambda b, i: (b, (i * qb_chunks - 1) % NCHUNK, 0)),
        ],
        out_specs=[
            pl.BlockSpec((1, QBLK, D), lambda b, i: (b, i, 0)),
            pl.BlockSpec((1, qb_chunks, BS), lambda b, i: (b, i, 0)),
        ],
        out_shape=[
            jax.ShapeDtypeStruct((B, TOT, D), jnp.float32),
            jax.ShapeDtypeStruct((B, NCHUNK, BS), jnp.float32),
        ],
    )(cmb, cmb)


EBLK = 1024


def _combine_body(og_ref, lg_ref, out_ref):
    og = og_ref[0]
    lg = lg_ref[0]
    m = jnp.max(lg, axis=0, keepdims=True)
    e = jnp.exp(lg - m)
    s = jnp.sum(e, axis=0, keepdims=True)
    w = e / s
    out_ref[0] = jnp.sum(og * w[:, :, None], axis=0)


def _combine(og, lg):
    return pl.pallas_call(
        _combine_body,
        grid=(B, S // EBLK),
        in_specs=[
            pl.BlockSpec((1, NH, EBLK, D), lambda b, i: (b, 0, i, 0)),
            pl.BlockSpec((1, NH, EBLK), lambda b, i: (b, 0, i)),
        ],
        out_specs=pl.BlockSpec((1, EBLK, D), lambda b, i: (b, i, 0)),
        out_shape=jax.ShapeDtypeStruct((B, S, D), jnp.float32),
    )(og, lg)


def kernel(qk, v, rotations):
    rot2 = rotations[0].reshape(D, NH * 32)
    pos = _hashpos(qk, rot2)
    pos_flat = pos.reshape(B, TOT)

    t_col = jnp.broadcast_to(
        jnp.arange(S, dtype=jnp.float32)[None, :, None], (B, S, 1))
    comb = jnp.concatenate(
        [qk, v, t_col,
         jnp.zeros((B, S, CW - 2 * D - 1), jnp.float32)], axis=-1)
    comb8 = jnp.tile(comb, (1, NH, 1))
    bidx = jnp.arange(B)[:, None]
    scmb = jnp.zeros((B, TOT, CW), jnp.float32).at[bidx, pos_flat].set(comb8)

    so, sl = _attn(scmb)
    sl_flat = sl.reshape(B, TOT)

    og = jnp.take_along_axis(
        so, jnp.broadcast_to(pos_flat[:, :, None], (B, TOT, D)), axis=1)
    lg = jnp.take_along_axis(sl_flat, pos_flat, axis=1)
    og = og.reshape(B, NH, S, D)
    lg = lg.reshape(B, NH, S)

    return _combine(og, lg)


# speedup vs baseline: 616.9457x; 4.3787x over previous
import functools

import jax
import jax.numpy as jnp
from jax import lax
from jax.experimental import pallas as pl
from jax.experimental.pallas import tpu as pltpu
from jax.experimental.pallas import tpu_sc as plsc

B, S, D = 16, 4096, 64
NH = 8
BS = 64
NB = S // BS
TOT = NH * S
NCHUNK = TOT // BS
CW = 128
OW = 128

NC, NS = 2, 16
NW = NC * NS
NTASK = B * NH
TPW = NTASK // NW
NJ = S // 128


def _hashpos_body(qk_ref, rot_ref, pos_ref):
    qk = qk_ref[0]
    rot = rot_ref[...]
    rotated = jax.lax.dot_general(
        qk, rot, (((1,), (0,)), ((), ())),
        preferred_element_type=jnp.float32)

    U = 128
    NCH = S // U
    io_r = jax.lax.broadcasted_iota(jnp.int32, (U, U), 0)
    io_c = jax.lax.broadcasted_iota(jnp.int32, (U, U), 1)
    trilB = jnp.broadcast_to(
        (io_r > io_c).astype(jnp.float32)[None], (NCH, U, U))
    ioc_r = jax.lax.broadcasted_iota(jnp.int32, (NCH, NCH), 0)
    ioc_c = jax.lax.broadcasted_iota(jnp.int32, (NCH, NCH), 1)
    tril_c = (ioc_r > ioc_c).astype(jnp.float32)
    iob_r = jax.lax.broadcasted_iota(jnp.int32, (NB, NB), 0)
    iob_c = jax.lax.broadcasted_iota(jnp.int32, (NB, NB), 1)
    upper_b = (iob_r < iob_c).astype(jnp.float32)

    rows = []
    for h in range(NH):
        r = rotated[:, h * 32:(h + 1) * 32]
        c = jnp.concatenate([r, -r], axis=-1)
        m = jnp.max(c, axis=-1, keepdims=True)
        i64 = jax.lax.broadcasted_iota(jnp.int32, (S, NB), 1)
        amax = jnp.min(jnp.where(c >= m, i64, NB), axis=-1, keepdims=True)
        oh = (i64 == amax).astype(jnp.float32)
        ohc = oh.reshape(NCH, U, NB)
        sc = jnp.sum(ohc, axis=1)
        pfx = jax.lax.dot_general(
            tril_c, sc, (((1,), (0,)), ((), ())),
            preferred_element_type=jnp.float32)
        cnt = jnp.sum(sc, axis=0, keepdims=True)
        excl = jax.lax.dot_general(
            cnt, upper_b, (((1,), (0,)), ((), ())),
            preferred_element_type=jnp.float32)
        mi = jax.lax.dot_general(
            trilB, ohc, (((2,), (1,)), ((0,), (0,))),
            preferred_element_type=jnp.float32)
        f = mi + pfx[:, None, :] + excl.reshape(1, 1, NB)
        posr = jnp.sum(f * ohc, axis=-1).reshape(S)
        rows.append(posr + h * S)
    pos_all = jnp.stack(rows, axis=0)
    pos_ref[0] = pos_all.astype(jnp.int32)


def _hashpos(qk, rot2):
    return pl.pallas_call(
        _hashpos_body,
        grid=(B,),
        in_specs=[
            pl.BlockSpec((1, S, D), lambda b: (b, 0, 0)),
            pl.BlockSpec((D, NH * 32), lambda b: (0, 0)),
        ],
        out_specs=pl.BlockSpec((1, NH, S), lambda b: (b, 0, 0)),
        out_shape=jax.ShapeDtypeStruct((B, NH, S), jnp.int32),
    )(qk, rot2)


@functools.lru_cache(maxsize=None)
def _make_sc_scatter():
    mesh = plsc.VectorSubcoreMesh(core_axis_name="c", subcore_axis_name="s")

    @functools.partial(
        pl.kernel,
        out_type=(
            jax.ShapeDtypeStruct((B * TOT, CW), jnp.float32),
            jax.ShapeDtypeStruct((NTASK, S), jnp.int32),
        ),
        mesh=mesh,
        compiler_params=pltpu.CompilerParams(needs_layout_passes=False),
        scratch_types=[
            pltpu.VMEM((NJ, 128), jnp.int32),
            pltpu.VMEM((S,), jnp.int32),
            pltpu.VMEM((S,), jnp.int32),
            pltpu.VMEM((128, CW), jnp.float32),
            pltpu.SemaphoreType.DMA,
        ],
    )
    def _sc_scatter(comb_hbm, posg_hbm, posl_hbm, out_hbm, st_hbm,
                    idx_v, idxl_v, st_seg, row_v, sem):
        wid = lax.axis_index("s") * NC + lax.axis_index("c")
        for k in range(TPW):
            tid = wid * TPW + k
            b = tid // NH
            pltpu.sync_copy(posg_hbm.at[tid], idx_v)
            pltpu.sync_copy(posl_hbm.at[tid], idxl_v)

            def st_body(i, carry):
                idx16 = idxl_v[pl.ds(i * 16, 16)]
                tv = lax.iota(jnp.int32, 16) + i * 16
                plsc.store_scatter(st_seg, [idx16], tv)
                return carry

            lax.fori_loop(0, S // 16, st_body, 0)
            pltpu.sync_copy(st_seg, st_hbm.at[tid])

            def row_body(j, carry):
                pltpu.sync_copy(comb_hbm.at[b, pl.ds(j * 128, 128)], row_v)
                pltpu.async_copy(row_v, out_hbm.at[idx_v.at[j]], sem).wait()
                return carry

            lax.fori_loop(0, NJ, row_body, 0)

    return _sc_scatter


@functools.lru_cache(maxsize=None)
def _make_sc_gather():
    mesh = plsc.VectorSubcoreMesh(core_axis_name="c", subcore_axis_name="s")

    @functools.partial(
        pl.kernel,
        out_type=jax.ShapeDtypeStruct((NTASK, S, OW), jnp.float32),
        mesh=mesh,
        compiler_params=pltpu.CompilerParams(needs_layout_passes=False),
        scratch_types=[
            pltpu.VMEM((NJ, 128), jnp.int32),
            pltpu.VMEM((128, OW), jnp.float32),
            pltpu.SemaphoreType.DMA,
        ],
    )
    def _sc_gather(so_hbm, posg_hbm, out_hbm, idx_v, row_v, sem):
        wid = lax.axis_index("s") * NC + lax.axis_index("c")
        for k in range(TPW):
            tid = wid * TPW + k
            pltpu.sync_copy(posg_hbm.at[tid], idx_v)

            def body(j, carry):
                pltpu.async_copy(so_hbm.at[idx_v.at[j]], row_v, sem).wait()
                pltpu.sync_copy(row_v, out_hbm.at[tid, pl.ds(j * 128, 128)])
                return carry

            lax.fori_loop(0, NJ, body, 0)

    return _sc_gather


QBLK = 1024
SUB = 256
KW = SUB + BS
NPROG = TOT // QBLK
QBC = QBLK // BS


SBC = SUB // BS


def _attn_body(cm_ref, ch_ref, stm_ref, sth_ref, so_ref):
    cm = cm_ref[0]
    ch = ch_ref[0]
    stm4 = stm_ref[0, 0].astype(jnp.float32)
    sth_row = sth_ref[0, 0].astype(jnp.float32)[QBC - 1]
    ext = jnp.concatenate([ch, cm], axis=0)
    ext_qk = ext[:, :D]
    ext_v = ext[:, D:2 * D]
    nrm = jnp.sqrt(jnp.sum(ext_qk * ext_qk, axis=-1, keepdims=True))
    ext_k = ext_qk / jnp.maximum(nrm, 1e-12)
    qm = cm[:, :D]
    sel = (jax.lax.broadcasted_iota(jnp.int32, (SUB, BS), 1)
           == jax.lax.broadcasted_iota(jnp.int32, (SUB, BS), 0) % BS)
    outs = []
    for sb in range(QBLK // SUB):
        q = qm[sb * SUB:(sb + 1) * SUB]
        k = ext_k[sb * SUB: sb * SUB + KW]
        v = ext_v[sb * SUB: sb * SUB + KW]
        dots = jax.lax.dot_general(
            q, k, (((1,), (1,)), ((), ())),
            preferred_element_type=jnp.float32) * (D ** -0.5)
        sub_stm = stm4[sb * SBC:(sb + 1) * SBC]
        rep = jnp.broadcast_to(
            sub_stm[:, None, :], (SBC, BS, BS)).reshape(SUB, BS)
        qt_col = jnp.sum(jnp.where(sel, rep, 0.0), axis=1,
                         keepdims=True)
        kt_rows = []
        for dd in range(SBC + 1):
            ci = sb * SBC + dd - 1
            row = sth_row if ci < 0 else stm4[ci]
            kt_rows.append(row[None, :])
        kt_row = jnp.concatenate(kt_rows, axis=1)
        dots = jnp.where(qt_col < kt_row, -1e9, dots)
        dots = jnp.where(qt_col == kt_row, -1e5, dots)
        qi = jax.lax.broadcasted_iota(jnp.int32, (SUB, KW), 0)
        kj = jax.lax.broadcasted_iota(jnp.int32, (SUB, KW), 1)
        qc = qi // BS
        kc = kj // BS - 1
        allowed = (kc == qc) | (kc == qc - 1)
        dots = jnp.where(allowed, dots, -1e9)
        m = jnp.max(dots, axis=-1, keepdims=True)
        p = jnp.exp(dots - m)
        s = jnp.sum(p, axis=-1, keepdims=True)
        o = jax.lax.dot_general(
            p, v, (((1,), (0,)), ((), ())),
            preferred_element_type=jnp.float32) / s
        lse = m + jnp.log(s)
        outs.append(jnp.concatenate(
            [o, lse, jnp.zeros((SUB, OW - D - 1), jnp.float32)], axis=-1))
    so_ref[0] = jnp.concatenate(outs, axis=0)


def _attn(scmb, st4):
    return pl.pallas_call(
        _attn_body,
        grid=(B, NPROG),
        in_specs=[
            pl.BlockSpec((1, QBLK, CW), lambda b, i: (b, i, 0)),
            pl.BlockSpec((1, BS, CW),
                         lambda b, i: (b, (i * QBC - 1) % NCHUNK, 0)),
            pl.BlockSpec((1, 1, QBC, BS), lambda b, i: (b, i, 0, 0)),
            pl.BlockSpec((1, 1, QBC, BS),
                         lambda b, i: (b, (i - 1) % NPROG, 0, 0)),
        ],
        out_specs=pl.BlockSpec((1, QBLK, OW), lambda b, i: (b, i, 0)),
        out_shape=jax.ShapeDtypeStruct((B, TOT, OW), jnp.float32),
    )(scmb, scmb, st4, st4)


EBLK = 1024


def _combine_body(og_ref, out_ref):
    og = og_ref[0]
    o = og[..., :D]
    lg = og[..., D]
    m = jnp.max(lg, axis=0, keepdims=True)
    e = jnp.exp(lg - m)
    s = jnp.sum(e, axis=0, keepdims=True)
    w = e / s
    out_ref[0] = jnp.sum(o * w[:, :, None], axis=0)


def _combine(og):
    return pl.pallas_call(
        _combine_body,
        grid=(B, S // EBLK),
        in_specs=[
            pl.BlockSpec((1, NH, EBLK, OW), lambda b, i: (b, 0, i, 0)),
        ],
        out_specs=pl.BlockSpec((1, EBLK, D), lambda b, i: (b, i, 0)),
        out_shape=jax.ShapeDtypeStruct((B, S, D), jnp.float32),
    )(og)


def kernel(qk, v, rotations):
    rot2 = rotations[0].reshape(D, NH * 32)
    pos = _hashpos(qk, rot2)

    pos_g = (pos + (jnp.arange(B, dtype=jnp.int32) * TOT)[:, None, None])
    pos_g = pos_g.reshape(NTASK, NJ, 128)
    pos_l = (pos % S).reshape(NTASK, S)

    comb = jnp.concatenate([qk, v], axis=-1)

    scmb, st = _make_sc_scatter()(comb, pos_g, pos_l)
    scmb = scmb.reshape(B, TOT, CW)
    st4 = st.reshape(B, NPROG, QBC, BS)

    so = _attn(scmb, st4)
    og = _make_sc_gather()(so.reshape(B * TOT, OW), pos_g)
    og = og.reshape(B, NH, S, OW)
    return _combine(og)


# speedup vs baseline: 689.3927x; 1.1174x over previous
import functools

import jax
import jax.numpy as jnp
from jax import lax
from jax.experimental import pallas as pl
from jax.experimental.pallas import tpu as pltpu
from jax.experimental.pallas import tpu_sc as plsc

B, S, D = 16, 4096, 64
NH = 8
BS = 64
NB = S // BS
TOT = NH * S
NCHUNK = TOT // BS
CW = 128
OW = 128

NC, NS = 2, 16
NW = NC * NS
NTASK = B * NH
TPW = NTASK // NW
NJ = S // 128


def _hashpos_body(qk_ref, rot_ref, pos_ref):
    qk = qk_ref[0]
    rot = rot_ref[...]
    rotated = jax.lax.dot_general(
        qk, rot, (((1,), (0,)), ((), ())),
        preferred_element_type=jnp.float32)

    U = 128
    NCH = S // U
    io_r = jax.lax.broadcasted_iota(jnp.int32, (U, U), 0)
    io_c = jax.lax.broadcasted_iota(jnp.int32, (U, U), 1)
    trilB = jnp.broadcast_to(
        (io_r > io_c).astype(jnp.float32)[None], (NCH, U, U))
    ioc_r = jax.lax.broadcasted_iota(jnp.int32, (NCH, NCH), 0)
    ioc_c = jax.lax.broadcasted_iota(jnp.int32, (NCH, NCH), 1)
    tril_c = (ioc_r > ioc_c).astype(jnp.float32)
    iob_r = jax.lax.broadcasted_iota(jnp.int32, (NB, NB), 0)
    iob_c = jax.lax.broadcasted_iota(jnp.int32, (NB, NB), 1)
    upper_b = (iob_r < iob_c).astype(jnp.float32)

    rows = []
    for h in range(NH):
        r = rotated[:, h * 32:(h + 1) * 32]
        c = jnp.concatenate([r, -r], axis=-1)
        m = jnp.max(c, axis=-1, keepdims=True)
        i64 = jax.lax.broadcasted_iota(jnp.int32, (S, NB), 1)
        amax = jnp.min(jnp.where(c >= m, i64, NB), axis=-1, keepdims=True)
        oh = (i64 == amax).astype(jnp.float32)
        ohc = oh.reshape(NCH, U, NB)
        sc = jnp.sum(ohc, axis=1)
        pfx = jax.lax.dot_general(
            tril_c, sc, (((1,), (0,)), ((), ())),
            preferred_element_type=jnp.float32)
        cnt = jnp.sum(sc, axis=0, keepdims=True)
        excl = jax.lax.dot_general(
            cnt, upper_b, (((1,), (0,)), ((), ())),
            preferred_element_type=jnp.float32)
        mi = jax.lax.dot_general(
            trilB.astype(jnp.bfloat16), ohc.astype(jnp.bfloat16),
            (((2,), (1,)), ((0,), (0,))),
            preferred_element_type=jnp.float32)
        f = mi + pfx[:, None, :] + excl.reshape(1, 1, NB)
        posr = jnp.sum(f * ohc, axis=-1).reshape(S)
        rows.append(posr + h * S)
    pos_all = jnp.stack(rows, axis=0)
    pos_ref[0] = pos_all.astype(jnp.int32)


def _hashpos(qk, rot2):
    return pl.pallas_call(
        _hashpos_body,
        grid=(B,),
        in_specs=[
            pl.BlockSpec((1, S, D), lambda b: (b, 0, 0)),
            pl.BlockSpec((D, NH * 32), lambda b: (0, 0)),
        ],
        out_specs=pl.BlockSpec((1, NH, S), lambda b: (b, 0, 0)),
        out_shape=jax.ShapeDtypeStruct((B, NH, S), jnp.int32),
    )(qk, rot2)


@functools.lru_cache(maxsize=None)
def _make_sc_scatter():
    mesh = plsc.VectorSubcoreMesh(core_axis_name="c", subcore_axis_name="s")

    @functools.partial(
        pl.kernel,
        out_type=(
            jax.ShapeDtypeStruct((B * TOT, CW), jnp.float32),
            jax.ShapeDtypeStruct((NTASK, S), jnp.int32),
        ),
        mesh=mesh,
        compiler_params=pltpu.CompilerParams(needs_layout_passes=False),
        scratch_types=[
            pltpu.VMEM((NJ, 128), jnp.int32),
            pltpu.VMEM((S,), jnp.int32),
            pltpu.VMEM((S,), jnp.int32),
            pltpu.VMEM((512, CW), jnp.float32),
            pltpu.SemaphoreType.DMA,
        ],
    )
    def _sc_scatter(comb_hbm, posg_hbm, posl_hbm, out_hbm, st_hbm,
                    idx_v, idxl_v, st_seg, row_v, sem):
        wid = lax.axis_index("s") * NC + lax.axis_index("c")
        for k in range(TPW):
            tid = wid * TPW + k
            b = tid // NH
            pltpu.sync_copy(posg_hbm.at[tid], idx_v)
            pltpu.sync_copy(posl_hbm.at[tid], idxl_v)

            def st_body(i, carry):
                idx16 = idxl_v[pl.ds(i * 16, 16)]
                tv = lax.iota(jnp.int32, 16) + i * 16
                plsc.store_scatter(st_seg, [idx16], tv)
                return carry

            lax.fori_loop(0, S // 16, st_body, 0)
            pltpu.sync_copy(st_seg, st_hbm.at[tid])

            def row_body(g, carry):
                pltpu.sync_copy(comb_hbm.at[b, pl.ds(g * 512, 512)], row_v)
                hs = [
                    pltpu.async_copy(
                        row_v.at[pl.ds(u * 128, 128)],
                        out_hbm.at[idx_v.at[g * 4 + u]], sem)
                    for u in range(4)
                ]
                for h in hs:
                    h.wait()
                return carry

            lax.fori_loop(0, NJ // 4, row_body, 0)

    return _sc_scatter


@functools.lru_cache(maxsize=None)
def _make_sc_gather():
    mesh = plsc.VectorSubcoreMesh(core_axis_name="c", subcore_axis_name="s")

    @functools.partial(
        pl.kernel,
        out_type=jax.ShapeDtypeStruct((NTASK, S, OW), jnp.float32),
        mesh=mesh,
        compiler_params=pltpu.CompilerParams(needs_layout_passes=False),
        scratch_types=[
            pltpu.VMEM((NJ, 128), jnp.int32),
            pltpu.VMEM((512, OW), jnp.float32),
            pltpu.SemaphoreType.DMA,
        ],
    )
    def _sc_gather(so_hbm, posg_hbm, out_hbm, idx_v, row_v, sem):
        wid = lax.axis_index("s") * NC + lax.axis_index("c")
        for k in range(TPW):
            tid = wid * TPW + k
            pltpu.sync_copy(posg_hbm.at[tid], idx_v)

            def body(g, carry):
                hs = [
                    pltpu.async_copy(
                        so_hbm.at[idx_v.at[g * 4 + u]],
                        row_v.at[pl.ds(u * 128, 128)], sem)
                    for u in range(4)
                ]
                for h in hs:
                    h.wait()
                pltpu.sync_copy(row_v, out_hbm.at[tid, pl.ds(g * 512, 512)])
                return carry

            lax.fori_loop(0, NJ // 4, body, 0)

    return _sc_gather


QBLK = 1024
SUB = 256
KW = SUB + BS
NPROG = TOT // QBLK
QBC = QBLK // BS


SBC = SUB // BS


def _attn_body(cm_ref, ch_ref, stm_ref, sth_ref, so_ref):
    cm = cm_ref[0]
    ch = ch_ref[0]
    stm4 = stm_ref[0, 0].astype(jnp.float32)
    sth_row = sth_ref[0, 0].astype(jnp.float32)[QBC - 1]
    ext = jnp.concatenate([ch, cm], axis=0)
    ext_qk = ext[:, :D]
    ext_v = ext[:, D:2 * D]
    nrm = jnp.sqrt(jnp.sum(ext_qk * ext_qk, axis=-1, keepdims=True))
    ext_k = ext_qk / jnp.maximum(nrm, 1e-12)
    qm = cm[:, :D]
    sel = (jax.lax.broadcasted_iota(jnp.int32, (SUB, BS), 1)
           == jax.lax.broadcasted_iota(jnp.int32, (SUB, BS), 0) % BS)
    outs = []
    for sb in range(QBLK // SUB):
        q = qm[sb * SUB:(sb + 1) * SUB]
        k = ext_k[sb * SUB: sb * SUB + KW]
        v = ext_v[sb * SUB: sb * SUB + KW]
        dots = jax.lax.dot_general(
            q.astype(jnp.bfloat16), k.astype(jnp.bfloat16),
            (((1,), (1,)), ((), ())),
            preferred_element_type=jnp.float32) * (D ** -0.5)
        sub_stm = stm4[sb * SBC:(sb + 1) * SBC]
        rep = jnp.broadcast_to(
            sub_stm[:, None, :], (SBC, BS, BS)).reshape(SUB, BS)
        qt_col = jnp.sum(jnp.where(sel, rep, 0.0), axis=1,
                         keepdims=True)
        kt_rows = []
        for dd in range(SBC + 1):
            ci = sb * SBC + dd - 1
            row = sth_row if ci < 0 else stm4[ci]
            kt_rows.append(row[None, :])
        kt_row = jnp.concatenate(kt_rows, axis=1)
        dots = jnp.where(qt_col < kt_row, -1e9, dots)
        dots = jnp.where(qt_col == kt_row, -1e5, dots)
        qi = jax.lax.broadcasted_iota(jnp.int32, (SUB, KW), 0)
        kj = jax.lax.broadcasted_iota(jnp.int32, (SUB, KW), 1)
        qc = qi // BS
        kc = kj // BS - 1
        allowed = (kc == qc) | (kc == qc - 1)
        dots = jnp.where(allowed, dots, -1e9)
        m = jnp.max(dots, axis=-1, keepdims=True)
        p = jnp.exp(dots - m)
        s = jnp.sum(p, axis=-1, keepdims=True)
        o = jax.lax.dot_general(
            p.astype(jnp.bfloat16), v.astype(jnp.bfloat16),
            (((1,), (0,)), ((), ())),
            preferred_element_type=jnp.float32) / s
        lse = m + jnp.log(s)
        outs.append(jnp.concatenate(
            [o, lse, jnp.zeros((SUB, OW - D - 1), jnp.float32)], axis=-1))
    so_ref[0] = jnp.concatenate(outs, axis=0)


def _attn(scmb, st4):
    return pl.pallas_call(
        _attn_body,
        grid=(B, NPROG),
        in_specs=[
            pl.BlockSpec((1, QBLK, CW), lambda b, i: (b, i, 0)),
            pl.BlockSpec((1, BS, CW),
                         lambda b, i: (b, (i * QBC - 1) % NCHUNK, 0)),
            pl.BlockSpec((1, 1, QBC, BS), lambda b, i: (b, i, 0, 0)),
            pl.BlockSpec((1, 1, QBC, BS),
                         lambda b, i: (b, (i - 1) % NPROG, 0, 0)),
        ],
        out_specs=pl.BlockSpec((1, QBLK, OW), lambda b, i: (b, i, 0)),
        out_shape=jax.ShapeDtypeStruct((B, TOT, OW), jnp.float32),
    )(scmb, scmb, st4, st4)


EBLK = 1024


def _combine_body(og_ref, out_ref):
    og = og_ref[0]
    o = og[..., :D]
    lg = og[..., D]
    m = jnp.max(lg, axis=0, keepdims=True)
    e = jnp.exp(lg - m)
    s = jnp.sum(e, axis=0, keepdims=True)
    w = e / s
    out_ref[0] = jnp.sum(o * w[:, :, None], axis=0)


def _combine(og):
    return pl.pallas_call(
        _combine_body,
        grid=(B, S // EBLK),
        in_specs=[
            pl.BlockSpec((1, NH, EBLK, OW), lambda b, i: (b, 0, i, 0)),
        ],
        out_specs=pl.BlockSpec((1, EBLK, D), lambda b, i: (b, i, 0)),
        out_shape=jax.ShapeDtypeStruct((B, S, D), jnp.float32),
    )(og)


def kernel(qk, v, rotations):
    rot2 = rotations[0].reshape(D, NH * 32)
    pos = _hashpos(qk, rot2)

    pos_g = (pos + (jnp.arange(B, dtype=jnp.int32) * TOT)[:, None, None])
    pos_g = pos_g.reshape(NTASK, NJ, 128)
    pos_l = (pos % S).reshape(NTASK, S)

    comb = jnp.concatenate([qk, v], axis=-1)

    scmb, st = _make_sc_scatter()(comb, pos_g, pos_l)
    scmb = scmb.reshape(B, TOT, CW)
    st4 = st.reshape(B, NPROG, QBC, BS)

    so = _attn(scmb, st4)
    og = _make_sc_gather()(so.reshape(B * TOT, OW), pos_g)
    og = og.reshape(B, NH, S, OW)
    return _combine(og)


# speedup vs baseline: 776.3833x; 1.1262x over previous
import functools

import jax
import jax.numpy as jnp
from jax import lax
from jax.experimental import pallas as pl
from jax.experimental.pallas import tpu as pltpu
from jax.experimental.pallas import tpu_sc as plsc

B, S, D = 16, 4096, 64
NH = 8
BS = 64
NB = S // BS
TOT = NH * S
NCHUNK = TOT // BS
CW = 128
OW = 128

NC, NS = 2, 16
NW = NC * NS
NTASK = B * NH
TPW = NTASK // NW
NJ = S // 128


def _hashpos_body(qk_ref, rot_ref, pos_ref):
    qk = qk_ref[0]
    rot = rot_ref[...]
    rotated = jax.lax.dot_general(
        qk, rot, (((1,), (0,)), ((), ())),
        preferred_element_type=jnp.float32)

    U = 128
    NCH = S // U
    io_r = jax.lax.broadcasted_iota(jnp.int32, (U, U), 0)
    io_c = jax.lax.broadcasted_iota(jnp.int32, (U, U), 1)
    trilB = jnp.broadcast_to(
        (io_r > io_c).astype(jnp.float32)[None], (NCH, U, U))
    ioc_r = jax.lax.broadcasted_iota(jnp.int32, (NCH, NCH), 0)
    ioc_c = jax.lax.broadcasted_iota(jnp.int32, (NCH, NCH), 1)
    tril_c = (ioc_r > ioc_c).astype(jnp.float32)
    iob_r = jax.lax.broadcasted_iota(jnp.int32, (NB, NB), 0)
    iob_c = jax.lax.broadcasted_iota(jnp.int32, (NB, NB), 1)
    upper_b = (iob_r < iob_c).astype(jnp.float32)

    rows = []
    for h in range(NH):
        r = rotated[:, h * 32:(h + 1) * 32]
        c = jnp.concatenate([r, -r], axis=-1)
        m = jnp.max(c, axis=-1, keepdims=True)
        i64 = jax.lax.broadcasted_iota(jnp.int32, (S, NB), 1)
        amax = jnp.min(jnp.where(c >= m, i64, NB), axis=-1, keepdims=True)
        oh = (i64 == amax).astype(jnp.float32)
        ohc = oh.reshape(NCH, U, NB)
        sc = jnp.sum(ohc, axis=1)
        pfx = jax.lax.dot_general(
            tril_c, sc, (((1,), (0,)), ((), ())),
            preferred_element_type=jnp.float32)
        cnt = jnp.sum(sc, axis=0, keepdims=True)
        excl = jax.lax.dot_general(
            cnt, upper_b, (((1,), (0,)), ((), ())),
            preferred_element_type=jnp.float32)
        mi = jax.lax.dot_general(
            trilB.astype(jnp.bfloat16), ohc.astype(jnp.bfloat16),
            (((2,), (1,)), ((0,), (0,))),
            preferred_element_type=jnp.float32)
        f = mi + pfx[:, None, :] + excl.reshape(1, 1, NB)
        posr = jnp.sum(f * ohc, axis=-1).reshape(S)
        rows.append(posr + h * S)
    pos_all = jnp.stack(rows, axis=0)
    pos_ref[0] = pos_all.astype(jnp.int32)


def _hashpos(qk, rot2):
    return pl.pallas_call(
        _hashpos_body,
        grid=(B,),
        in_specs=[
            pl.BlockSpec((1, S, D), lambda b: (b, 0, 0)),
            pl.BlockSpec((D, NH * 32), lambda b: (0, 0)),
        ],
        out_specs=pl.BlockSpec((1, NH, S), lambda b: (b, 0, 0)),
        out_shape=jax.ShapeDtypeStruct((B, NH, S), jnp.int32),
    )(qk, rot2)


@functools.lru_cache(maxsize=None)
def _make_sc_scatter():
    mesh = plsc.VectorSubcoreMesh(core_axis_name="c", subcore_axis_name="s")

    @functools.partial(
        pl.kernel,
        out_type=(
            jax.ShapeDtypeStruct((B * TOT, CW), jnp.float32),
            jax.ShapeDtypeStruct((NTASK, S), jnp.int32),
        ),
        mesh=mesh,
        compiler_params=pltpu.CompilerParams(needs_layout_passes=False),
        scratch_types=[
            pltpu.VMEM((NJ, 128), jnp.int32),
            pltpu.VMEM((S,), jnp.int32),
            pltpu.VMEM((S,), jnp.int32),
            pltpu.VMEM((512, CW), jnp.float32),
            pltpu.SemaphoreType.DMA,
        ],
    )
    def _sc_scatter(comb_hbm, posg_hbm, posl_hbm, out_hbm, st_hbm,
                    idx_v, idxl_v, st_seg, row_v, sem):
        wid = lax.axis_index("s") * NC + lax.axis_index("c")
        for k in range(TPW):
            tid = wid * TPW + k
            b = tid // NH
            pltpu.sync_copy(posg_hbm.at[tid], idx_v)
            pltpu.sync_copy(posl_hbm.at[tid], idxl_v)

            def st_body(i, carry):
                idx16 = idxl_v[pl.ds(i * 16, 16)]
                tv = lax.iota(jnp.int32, 16) + i * 16
                plsc.store_scatter(st_seg, [idx16], tv)
                return carry

            lax.fori_loop(0, S // 16, st_body, 0)
            pltpu.sync_copy(st_seg, st_hbm.at[tid])

            def row_body(g, carry):
                pltpu.sync_copy(comb_hbm.at[b, pl.ds(g * 512, 512)], row_v)
                hs = [
                    pltpu.async_copy(
                        row_v.at[pl.ds(u * 128, 128)],
                        out_hbm.at[idx_v.at[g * 4 + u]], sem)
                    for u in range(4)
                ]
                for h in hs:
                    h.wait()
                return carry

            lax.fori_loop(0, NJ // 4, row_body, 0)

    return _sc_scatter


@functools.lru_cache(maxsize=None)
def _make_sc_gather():
    mesh = plsc.VectorSubcoreMesh(core_axis_name="c", subcore_axis_name="s")

    @functools.partial(
        pl.kernel,
        out_type=jax.ShapeDtypeStruct((NTASK, S, OW), jnp.float32),
        mesh=mesh,
        compiler_params=pltpu.CompilerParams(needs_layout_passes=False),
        scratch_types=[
            pltpu.VMEM((NJ, 128), jnp.int32),
            pltpu.VMEM((512, OW), jnp.float32),
            pltpu.SemaphoreType.DMA,
        ],
    )
    def _sc_gather(so_hbm, posg_hbm, out_hbm, idx_v, row_v, sem):
        wid = lax.axis_index("s") * NC + lax.axis_index("c")
        for k in range(TPW):
            tid = wid * TPW + k
            pltpu.sync_copy(posg_hbm.at[tid], idx_v)

            def body(g, carry):
                hs = [
                    pltpu.async_copy(
                        so_hbm.at[idx_v.at[g * 4 + u]],
                        row_v.at[pl.ds(u * 128, 128)], sem)
                    for u in range(4)
                ]
                for h in hs:
                    h.wait()
                pltpu.sync_copy(row_v, out_hbm.at[tid, pl.ds(g * 512, 512)])
                return carry

            lax.fori_loop(0, NJ // 4, body, 0)

    return _sc_gather


QBLK = 1024
SUB = 256
KW = SUB + BS
NPROG = TOT // QBLK
QBC = QBLK // BS


def _attn_body(cm_ref, ch_ref, stm_ref, sth_ref, so_ref):
    cm = cm_ref[0]
    ch = ch_ref[0]
    stm4 = stm_ref[0, 0].astype(jnp.float32)
    sth_row = sth_ref[0, 0].astype(jnp.float32)[QBC - 1:QBC]
    ext = jnp.concatenate([ch, cm], axis=0)
    ext_qk = ext[:, :D]
    ext_v = ext[:, D:2 * D]
    nrm = jnp.sqrt(jnp.sum(ext_qk * ext_qk, axis=-1, keepdims=True))
    ext_k = (ext_qk / jnp.maximum(nrm, 1e-12)).astype(jnp.bfloat16)
    ext_vb = ext_v.astype(jnp.bfloat16)

    q3 = cm[:, :D].astype(jnp.bfloat16).reshape(QBC, BS, D)
    k3 = jnp.concatenate(
        [ext_k[:QBLK].reshape(QBC, BS, D),
         ext_k[BS:].reshape(QBC, BS, D)], axis=1)
    v3 = jnp.concatenate(
        [ext_vb[:QBLK].reshape(QBC, BS, D),
         ext_vb[BS:].reshape(QBC, BS, D)], axis=1)
    dots = jax.lax.dot_general(
        q3, k3, (((2,), (2,)), ((0,), (0,))),
        preferred_element_type=jnp.float32) * (D ** -0.5)

    sel = (jax.lax.broadcasted_iota(jnp.int32, (BS, BS), 0)
           == jax.lax.broadcasted_iota(jnp.int32, (BS, BS), 1))
    rep = jnp.broadcast_to(stm4[:, None, :], (QBC, BS, BS))
    qt3 = jnp.sum(jnp.where(sel[None], rep, 0.0), axis=2,
                  keepdims=True)
    ta = jnp.concatenate([sth_row, stm4[:QBC - 1]], axis=0)
    kt3 = jnp.concatenate([ta, stm4], axis=1)[:, None, :]

    dots = jnp.where(qt3 < kt3, -1e9,
                     jnp.where(qt3 == kt3, -1e5, dots))
    m = jnp.max(dots, axis=-1, keepdims=True)
    p = jnp.exp(dots - m)
    s = jnp.sum(p, axis=-1, keepdims=True)
    o3 = jax.lax.dot_general(
        p.astype(jnp.bfloat16), v3, (((2,), (1,)), ((0,), (0,))),
        preferred_element_type=jnp.float32) / s
    lse = (m + jnp.log(s)).reshape(QBLK, 1)
    o = o3.reshape(QBLK, D)
    so_ref[0] = jnp.concatenate(
        [o, lse, jnp.zeros((QBLK, OW - D - 1), jnp.float32)], axis=-1)


def _attn(scmb, st4):
    return pl.pallas_call(
        _attn_body,
        grid=(B, NPROG),
        in_specs=[
            pl.BlockSpec((1, QBLK, CW), lambda b, i: (b, i, 0)),
            pl.BlockSpec((1, BS, CW),
                         lambda b, i: (b, (i * QBC - 1) % NCHUNK, 0)),
            pl.BlockSpec((1, 1, QBC, BS), lambda b, i: (b, i, 0, 0)),
            pl.BlockSpec((1, 1, QBC, BS),
                         lambda b, i: (b, (i - 1) % NPROG, 0, 0)),
        ],
        out_specs=pl.BlockSpec((1, QBLK, OW), lambda b, i: (b, i, 0)),
        out_shape=jax.ShapeDtypeStruct((B, TOT, OW), jnp.float32),
    )(scmb, scmb, st4, st4)


EBLK = 1024


def _combine_body(og_ref, out_ref):
    og = og_ref[0]
    o = og[..., :D]
    lg = og[..., D]
    m = jnp.max(lg, axis=0, keepdims=True)
    e = jnp.exp(lg - m)
    s = jnp.sum(e, axis=0, keepdims=True)
    w = e / s
    out_ref[0] = jnp.sum(o * w[:, :, None], axis=0)


def _combine(og):
    return pl.pallas_call(
        _combine_body,
        grid=(B, S // EBLK),
        in_specs=[
            pl.BlockSpec((1, NH, EBLK, OW), lambda b, i: (b, 0, i, 0)),
        ],
        out_specs=pl.BlockSpec((1, EBLK, D), lambda b, i: (b, i, 0)),
        out_shape=jax.ShapeDtypeStruct((B, S, D), jnp.float32),
    )(og)


def kernel(qk, v, rotations):
    rot2 = rotations[0].reshape(D, NH * 32)
    pos = _hashpos(qk, rot2)

    pos_g = (pos + (jnp.arange(B, dtype=jnp.int32) * TOT)[:, None, None])
    pos_g = pos_g.reshape(NTASK, NJ, 128)
    pos_l = (pos % S).reshape(NTASK, S)

    comb = jnp.concatenate([qk, v], axis=-1)

    scmb, st = _make_sc_scatter()(comb, pos_g, pos_l)
    scmb = scmb.reshape(B, TOT, CW)
    st4 = st.reshape(B, NPROG, QBC, BS)

    so = _attn(scmb, st4)
    og = _make_sc_gather()(so.reshape(B * TOT, OW), pos_g)
    og = og.reshape(B, NH, S, OW)
    return _combine(og)


# speedup vs baseline: 851.5778x; 1.0969x over previous
import functools

import jax
import jax.numpy as jnp
from jax import lax
from jax.experimental import pallas as pl
from jax.experimental.pallas import tpu as pltpu
from jax.experimental.pallas import tpu_sc as plsc

B, S, D = 16, 4096, 64
NH = 8
BS = 64
NB = S // BS
TOT = NH * S
NCHUNK = TOT // BS
CW = 128
OW = 128

NC, NS = 2, 16
NW = NC * NS
NTASK = B * NH
TPW = NTASK // NW
NJ = S // 128


def _hashpos_body(qk_ref, rot_ref, pos_ref):
    qk = qk_ref[0]
    rot = rot_ref[...]
    rote = jax.lax.dot_general(
        qk, rot, (((1,), (0,)), ((), ())),
        preferred_element_type=jnp.float32)

    U = 128
    NCH = S // U
    io_r = jax.lax.broadcasted_iota(jnp.int32, (U, U), 0)
    io_c = jax.lax.broadcasted_iota(jnp.int32, (U, U), 1)
    trilB = jnp.broadcast_to(
        (io_r > io_c).astype(jnp.bfloat16)[None], (NCH, U, U))
    ioc_r = jax.lax.broadcasted_iota(jnp.int32, (NCH, NCH), 0)
    ioc_c = jax.lax.broadcasted_iota(jnp.int32, (NCH, NCH), 1)
    tril_c = (ioc_r > ioc_c).astype(jnp.float32)
    upper2 = ((io_r < io_c) & (io_r // NB == io_c // NB)
              ).astype(jnp.float32)

    rows = []
    for hp in range(NH // 2):
        ohs = []
        for hh in range(2):
            h = 2 * hp + hh
            c = rote[:, NB * h: NB * (h + 1)]
            m = jnp.max(c, axis=-1, keepdims=True)
            i64 = jax.lax.broadcasted_iota(jnp.int32, (S, NB), 1)
            amax = jnp.min(jnp.where(c >= m, i64, NB), axis=-1,
                           keepdims=True)
            ohs.append((i64 == amax).astype(jnp.float32))
        ohp = jnp.concatenate(ohs, axis=-1)
        ohcp = ohp.reshape(NCH, U, 2 * NB)
        scp = jnp.sum(ohcp, axis=1)
        pfxp = jax.lax.dot_general(
            tril_c, scp, (((1,), (0,)), ((), ())),
            preferred_element_type=jnp.float32)
        cntp = jnp.sum(scp, axis=0, keepdims=True)
        exclp = jax.lax.dot_general(
            cntp, upper2, (((1,), (0,)), ((), ())),
            preferred_element_type=jnp.float32)
        mip = jax.lax.dot_general(
            trilB, ohcp.astype(jnp.bfloat16),
            (((2,), (1,)), ((0,), (0,))),
            preferred_element_type=jnp.float32)
        fp = mip + pfxp[:, None, :] + exclp.reshape(1, 1, 2 * NB)
        g = fp * ohcp
        pa = jnp.sum(g[:, :, :NB], axis=-1).reshape(S)
        pb = jnp.sum(g[:, :, NB:], axis=-1).reshape(S)
        rows.append(pa + (2 * hp) * S)
        rows.append(pb + (2 * hp + 1) * S)
    pos_all = jnp.stack(rows, axis=0)
    pos_ref[0] = pos_all.astype(jnp.int32)


def _hashpos(qk, rot2):
    return pl.pallas_call(
        _hashpos_body,
        grid=(B,),
        in_specs=[
            pl.BlockSpec((1, S, D), lambda b: (b, 0, 0)),
            pl.BlockSpec((D, NH * 64), lambda b: (0, 0)),
        ],
        out_specs=pl.BlockSpec((1, NH, S), lambda b: (b, 0, 0)),
        out_shape=jax.ShapeDtypeStruct((B, NH, S), jnp.int32),
    )(qk, rot2)


@functools.lru_cache(maxsize=None)
def _make_sc_scatter():
    mesh = plsc.VectorSubcoreMesh(core_axis_name="c", subcore_axis_name="s")

    @functools.partial(
        pl.kernel,
        out_type=(
            jax.ShapeDtypeStruct((B * TOT, CW), jnp.float32),
            jax.ShapeDtypeStruct((NTASK, S), jnp.int32),
        ),
        mesh=mesh,
        compiler_params=pltpu.CompilerParams(needs_layout_passes=False),
        scratch_types=[
            pltpu.VMEM((NJ, 128), jnp.int32),
            pltpu.VMEM((S,), jnp.int32),
            pltpu.VMEM((S,), jnp.int32),
            pltpu.VMEM((512, CW), jnp.float32),
            pltpu.SemaphoreType.DMA,
        ],
    )
    def _sc_scatter(comb_hbm, posg_hbm, posl_hbm, out_hbm, st_hbm,
                    idx_v, idxl_v, st_seg, row_v, sem):
        wid = lax.axis_index("s") * NC + lax.axis_index("c")
        for k in range(TPW):
            tid = wid * TPW + k
            b = tid // NH
            pltpu.sync_copy(posg_hbm.at[tid], idx_v)
            pltpu.sync_copy(posl_hbm.at[tid], idxl_v)

            def st_body(i, carry):
                idx16 = idxl_v[pl.ds(i * 16, 16)]
                tv = lax.iota(jnp.int32, 16) + i * 16
                plsc.store_scatter(st_seg, [idx16], tv)
                return carry

            lax.fori_loop(0, S // 16, st_body, 0)
            pltpu.sync_copy(st_seg, st_hbm.at[tid])

            def row_body(g, carry):
                pltpu.sync_copy(comb_hbm.at[b, pl.ds(g * 512, 512)], row_v)
                hs = [
                    pltpu.async_copy(
                        row_v.at[pl.ds(u * 128, 128)],
                        out_hbm.at[idx_v.at[g * 4 + u]], sem)
                    for u in range(4)
                ]
                for h in hs:
                    h.wait()
                return carry

            lax.fori_loop(0, NJ // 4, row_body, 0)

    return _sc_scatter


@functools.lru_cache(maxsize=None)
def _make_sc_gather():
    mesh = plsc.VectorSubcoreMesh(core_axis_name="c", subcore_axis_name="s")

    @functools.partial(
        pl.kernel,
        out_type=jax.ShapeDtypeStruct((NTASK, S, OW), jnp.float32),
        mesh=mesh,
        compiler_params=pltpu.CompilerParams(needs_layout_passes=False),
        scratch_types=[
            pltpu.VMEM((NJ, 128), jnp.int32),
            pltpu.VMEM((512, OW), jnp.float32),
            pltpu.SemaphoreType.DMA,
        ],
    )
    def _sc_gather(so_hbm, posg_hbm, out_hbm, idx_v, row_v, sem):
        wid = lax.axis_index("s") * NC + lax.axis_index("c")
        for k in range(TPW):
            tid = wid * TPW + k
            pltpu.sync_copy(posg_hbm.at[tid], idx_v)

            def body(g, carry):
                hs = [
                    pltpu.async_copy(
                        so_hbm.at[idx_v.at[g * 4 + u]],
                        row_v.at[pl.ds(u * 128, 128)], sem)
                    for u in range(4)
                ]
                for h in hs:
                    h.wait()
                pltpu.sync_copy(row_v, out_hbm.at[tid, pl.ds(g * 512, 512)])
                return carry

            lax.fori_loop(0, NJ // 4, body, 0)

    return _sc_gather


QBLK = 1024
SUB = 256
KW = SUB + BS
NPROG = TOT // QBLK
QBC = QBLK // BS


def _attn_body(cm_ref, ch_ref, stm_ref, sth_ref, so_ref):
    cm = cm_ref[0]
    ch = ch_ref[0]
    stm4 = stm_ref[0, 0].astype(jnp.float32)
    sth_row = sth_ref[0, 0].astype(jnp.float32)[QBC - 1:QBC]
    ext = jnp.concatenate([ch, cm], axis=0)
    ext_qk = ext[:, :D]
    ext_v = ext[:, D:2 * D]
    nrm = jnp.sqrt(jnp.sum(ext_qk * ext_qk, axis=-1, keepdims=True))
    ext_k = (ext_qk / jnp.maximum(nrm, 1e-12)).astype(jnp.bfloat16)
    ext_vb = ext_v.astype(jnp.bfloat16)

    q3 = cm[:, :D].astype(jnp.bfloat16).reshape(QBC, BS, D)
    k3 = jnp.concatenate(
        [ext_k[:QBLK].reshape(QBC, BS, D),
         ext_k[BS:].reshape(QBC, BS, D)], axis=1)
    v3 = jnp.concatenate(
        [ext_vb[:QBLK].reshape(QBC, BS, D),
         ext_vb[BS:].reshape(QBC, BS, D)], axis=1)
    dots = jax.lax.dot_general(
        q3, k3, (((2,), (2,)), ((0,), (0,))),
        preferred_element_type=jnp.float32) * (D ** -0.5)

    sel = (jax.lax.broadcasted_iota(jnp.int32, (BS, BS), 0)
           == jax.lax.broadcasted_iota(jnp.int32, (BS, BS), 1))
    rep = jnp.broadcast_to(stm4[:, None, :], (QBC, BS, BS))
    qt3 = jnp.sum(jnp.where(sel[None], rep, 0.0), axis=2,
                  keepdims=True)
    ta = jnp.concatenate([sth_row, stm4[:QBC - 1]], axis=0)
    kt3 = jnp.concatenate([ta, stm4], axis=1)[:, None, :]

    dots = jnp.where(qt3 < kt3, -1e9,
                     jnp.where(qt3 == kt3, -1e5, dots))
    m = jnp.max(dots, axis=-1, keepdims=True)
    p = jnp.exp(dots - m)
    s = jnp.sum(p, axis=-1, keepdims=True)
    o3 = jax.lax.dot_general(
        p.astype(jnp.bfloat16), v3, (((2,), (1,)), ((0,), (0,))),
        preferred_element_type=jnp.float32) / s
    lse = (m + jnp.log(s)).reshape(QBLK, 1)
    o = o3.reshape(QBLK, D)
    so_ref[0] = jnp.concatenate(
        [o, lse, jnp.zeros((QBLK, OW - D - 1), jnp.float32)], axis=-1)


def _attn(scmb, st4):
    return pl.pallas_call(
        _attn_body,
        grid=(B, NPROG),
        in_specs=[
            pl.BlockSpec((1, QBLK, CW), lambda b, i: (b, i, 0)),
            pl.BlockSpec((1, BS, CW),
                         lambda b, i: (b, (i * QBC - 1) % NCHUNK, 0)),
            pl.BlockSpec((1, 1, QBC, BS), lambda b, i: (b, i, 0, 0)),
            pl.BlockSpec((1, 1, QBC, BS),
                         lambda b, i: (b, (i - 1) % NPROG, 0, 0)),
        ],
        out_specs=pl.BlockSpec((1, QBLK, OW), lambda b, i: (b, i, 0)),
        out_shape=jax.ShapeDtypeStruct((B, TOT, OW), jnp.float32),
    )(scmb, scmb, st4, st4)


EBLK = 1024


def _combine_body(og_ref, out_ref):
    og = og_ref[0]
    o = og[..., :D]
    lg = og[..., D]
    m = jnp.max(lg, axis=0, keepdims=True)
    e = jnp.exp(lg - m)
    s = jnp.sum(e, axis=0, keepdims=True)
    w = e / s
    out_ref[0] = jnp.sum(o * w[:, :, None], axis=0)


def _combine(og):
    return pl.pallas_call(
        _combine_body,
        grid=(B, S // EBLK),
        in_specs=[
            pl.BlockSpec((1, NH, EBLK, OW), lambda b, i: (b, 0, i, 0)),
        ],
        out_specs=pl.BlockSpec((1, EBLK, D), lambda b, i: (b, i, 0)),
        out_shape=jax.ShapeDtypeStruct((B, S, D), jnp.float32),
    )(og)


def kernel(qk, v, rotations):
    rot3 = rotations[0].reshape(D, NH, 32)
    rot2 = jnp.concatenate([rot3, -rot3], axis=-1).reshape(D, NH * 64)
    pos = _hashpos(qk, rot2)

    pos_g = (pos + (jnp.arange(B, dtype=jnp.int32) * TOT)[:, None, None])
    pos_g = pos_g.reshape(NTASK, NJ, 128)
    pos_l = (pos % S).reshape(NTASK, S)

    comb = jnp.concatenate([qk, v], axis=-1)

    scmb, st = _make_sc_scatter()(comb, pos_g, pos_l)
    scmb = scmb.reshape(B, TOT, CW)
    st4 = st.reshape(B, NPROG, QBC, BS)

    so = _attn(scmb, st4)
    og = _make_sc_gather()(so.reshape(B * TOT, OW), pos_g)
    og = og.reshape(B, NH, S, OW)
    return _combine(og)
